# baseline (speedup 1.0000x reference)
import functools

import jax
import jax.numpy as jnp
from jax import lax
from jax.experimental import pallas as pl
from jax.experimental.pallas import tpu as pltpu
from jax.experimental.pallas import tpu_sc as plsc

N = 10000
E = 320000
D = 128
N_PAD = N
BR = 1000
NC, NS = 2, 16
NW = NC * NS
EPT = E // NW
K = 80
NCH = EPT // K
PH = 5
CPP = NCH // PH
ZR = 5

f32 = jnp.float32


@functools.lru_cache(maxsize=None)
def _make_sc_segsum(with_deg: bool):
    out_type = [jax.ShapeDtypeStruct((NC, N_PAD, D), f32)]
    zr = ZR
    scratch = [
        pltpu.VMEM_SHARED((N_PAD, D), f32),
        pltpu.VMEM((zr, D), f32),
        pltpu.SemaphoreType.DMA,
        pltpu.VMEM((K, D), f32),
        pltpu.VMEM((K, D), f32),
        pltpu.VMEM((K, D), f32),
        pltpu.SemaphoreType.DMA,
        pltpu.SemaphoreType.DMA,
        pltpu.SemaphoreType.DMA,
        pltpu.SemaphoreType.DMA,
        pltpu.SemaphoreType.DMA,
        pltpu.SemaphoreType.DMA,
    ]
    if with_deg:
        out_type.append(jax.ShapeDtypeStruct((NC, N_PAD, 16), f32))
        scratch += [
            pltpu.VMEM((CPP, K), jnp.int32),
            pltpu.VMEM((CPP, K), jnp.int32),
            pltpu.SemaphoreType.DMA,
            pltpu.VMEM_SHARED((N_PAD, 16), f32),
            pltpu.VMEM((ZR, 16), f32),
            pltpu.VMEM((K, 16), f32),
            pltpu.SemaphoreType.DMA,
        ]
    else:
        scratch += [
            pltpu.VMEM((CPP, K), jnp.int32),
            pltpu.VMEM((CPP, K), jnp.int32),
            pltpu.VMEM((CPP, K), jnp.int32),
            pltpu.VMEM((CPP, K), jnp.int32),
            pltpu.SemaphoreType.DMA,
            pltpu.SemaphoreType.DMA,
            pltpu.VMEM((K, D), f32),
            pltpu.SemaphoreType.DMA,
            pltpu.SemaphoreType.DMA,
        ]

    mesh = plsc.VectorSubcoreMesh(core_axis_name="c", subcore_axis_name="s",
                                  num_cores=NC, num_subcores=NS)

    @functools.partial(
        pl.kernel, out_type=out_type, mesh=mesh, scratch_types=scratch,
        compiler_params=pltpu.CompilerParams(use_tc_tiling_on_sc=False))
    def sc_kernel(p_hbm, adj_hbm, *refs):
        if with_deg:
            (agg_hbm, deg_hbm, acc, zbuf, zsem, rows0, rows1, rows2,
             sem0, sem1, sem2, ssem0, ssem1, ssem2,
             sidx0, didx0, isem0, dacc, dzbuf, ones, osem) = refs
            islot = [(sidx0, didx0, isem0)]
        else:
            (agg_hbm, acc, zbuf, zsem, rows0, rows1, rows2,
             sem0, sem1, sem2, ssem0, ssem1, ssem2,
             sidx0, didx0, sidx1, didx1, isem0, isem1,
             rows3, sem3, ssem3) = refs
            islot = [(sidx0, didx0, isem0), (sidx1, didx1, isem1)]
        cid = lax.axis_index("c")
        sid = lax.axis_index("s")
        wid = sid * NC + cid
        rpt = N_PAD // NS
        nsl = len(islot)

        def issue_idx(ph):
            si, di, isem = islot[ph % nsl]
            r0 = wid * NCH + ph * CPP
            pltpu.async_copy(adj_hbm.at[0, pl.ds(r0, CPP)], si, isem)
            pltpu.async_copy(adj_hbm.at[1, pl.ds(r0, CPP)], di, isem)

        def wait_idx(ph):
            si, di, isem = islot[ph % nsl]
            pltpu.make_async_copy(adj_hbm.at[0, pl.ds(0, CPP)], si,
                                  isem).wait()
            pltpu.make_async_copy(adj_hbm.at[1, pl.ds(0, CPP)], di,
                                  isem).wait()

        issue_idx(0)

        z16 = jnp.zeros((16,), f32)

        def zrow(r, _):
            for j in range(D // 16):
                zbuf[r, pl.ds(j * 16, 16)] = z16
            return 0
        lax.fori_loop(0, zr, zrow, 0)

        def zacc(i, _):
            pltpu.async_copy(zbuf, acc.at[pl.ds(sid * rpt + i * zr, zr)],
                             zsem)
            return 0
        lax.fori_loop(0, rpt // zr, zacc, 0)

        if with_deg:
            def zdrow(r, _):
                dzbuf[r, :] = z16
                return 0
            lax.fori_loop(0, zr, zdrow, 0)

            def zdacc(i, _):
                pltpu.async_copy(dzbuf,
                                 dacc.at[pl.ds(sid * rpt + i * zr, zr)],
                                 zsem)
                return 0
            lax.fori_loop(0, rpt // zr, zdacc, 0)
            o16 = jnp.ones((16,), f32)

            def orow(r, _):
                ones[r, :] = o16
                return 0
            lax.fori_loop(0, K, orow, 0)

        wait_idx(0)
        sidx_p0 = islot[0][0]
        pltpu.async_copy(p_hbm.at[sidx_p0.at[0]], rows0, sem0)
        pltpu.async_copy(p_hbm.at[sidx_p0.at[1]], rows1, sem1)
        if not with_deg:
            pltpu.async_copy(p_hbm.at[sidx_p0.at[2]], rows2, sem2)

        def zdrain(i, _):
            pltpu.make_async_copy(zbuf, acc.at[pl.ds(0, zr)], zsem).wait()
            return 0
        lax.fori_loop(0, rpt // zr, zdrain, 0)
        if with_deg:
            def zddrain(i, _):
                pltpu.make_async_copy(dzbuf, dacc.at[pl.ds(0, zr)],
                                      zsem).wait()
                return 0
            lax.fori_loop(0, rpt // zr, zddrain, 0)

        plsc.subcore_barrier()

        nb = (CPP - 1) // 3
        assert CPP == 3 * nb + 1 and CPP == 4 * (CPP // 4) + 1
        for ph in range(PH):
            sidx, didx, _ = islot[ph % nsl]
            if nsl == 2 and ph + 1 < PH:
                issue_idx(ph + 1)
            if ph > 0:
                if nsl == 1:
                    issue_idx(ph)
                wait_idx(ph)
                pltpu.async_copy(p_hbm.at[sidx.at[0]], rows0, sem0)
                pltpu.async_copy(p_hbm.at[sidx.at[1]], rows1, sem1)
                if not with_deg:
                    pltpu.async_copy(p_hbm.at[sidx.at[2]], rows2, sem2)

            def wait_g(rows, sem):
                pltpu.make_async_copy(p_hbm.at[sidx.at[0]], rows, sem).wait()

            def iscat(rows, c, ssem):
                pltpu.async_copy(rows, acc.at[didx.at[c]], ssem, add=True)
                if with_deg:
                    pltpu.async_copy(ones, dacc.at[didx.at[c]], osem,
                                     add=True)

            def wait_s(rows, ssem):
                pltpu.make_async_copy(rows, acc.at[didx.at[0]], ssem).wait()

            if with_deg:
                def ring(i, _):
                    c = 3 * i
                    wait_g(rows0, sem0)
                    iscat(rows0, c, ssem0)
                    pltpu.async_copy(p_hbm.at[sidx.at[c + 2]], rows2, sem2)
                    wait_g(rows1, sem1)
                    iscat(rows1, c + 1, ssem1)
                    wait_s(rows0, ssem0)
                    pltpu.async_copy(p_hbm.at[sidx.at[c + 3]], rows0, sem0)
                    wait_g(rows2, sem2)
                    iscat(rows2, c + 2, ssem2)
                    wait_s(rows1, ssem1)

                    @pl.when(c + 4 < CPP)
                    def _():
                        pltpu.async_copy(p_hbm.at[sidx.at[c + 4]], rows1,
                                         sem1)
                    wait_s(rows2, ssem2)
                    return 0
                lax.fori_loop(0, nb, ring, 0)
                wait_g(rows0, sem0)
                pltpu.sync_copy(rows0, acc.at[didx.at[CPP - 1]], add=True)
                pltpu.sync_copy(ones, dacc.at[didx.at[CPP - 1]], add=True)

                def odrain(i, _):
                    pltpu.make_async_copy(ones, dacc.at[didx.at[0]],
                                          osem).wait()
                    return 0
                lax.fori_loop(0, 3 * nb, odrain, 0)
            else:
                def ring4(i, _):
                    c = 4 * i
                    wait_g(rows0, sem0)
                    iscat(rows0, c, ssem0)
                    pltpu.async_copy(p_hbm.at[sidx.at[c + 3]], rows3, sem3)
                    wait_g(rows1, sem1)
                    iscat(rows1, c + 1, ssem1)
                    wait_s(rows0, ssem0)
                    pltpu.async_copy(p_hbm.at[sidx.at[c + 4]], rows0, sem0)
                    wait_g(rows2, sem2)
                    iscat(rows2, c + 2, ssem2)
                    wait_s(rows1, ssem1)

                    @pl.when(c + 5 < CPP)
                    def _():
                        pltpu.async_copy(p_hbm.at[sidx.at[c + 5]], rows1,
                                         sem1)
                    wait_g(rows3, sem3)
                    iscat(rows3, c + 3, ssem3)
                    wait_s(rows2, ssem2)

                    @pl.when(c + 6 < CPP)
                    def _():
                        pltpu.async_copy(p_hbm.at[sidx.at[c + 6]], rows2,
                                         sem2)
                    wait_s(rows3, ssem3)
                    return 0
                lax.fori_loop(0, CPP // 4, ring4, 0)
                wait_g(rows0, sem0)
                pltpu.sync_copy(rows0, acc.at[didx.at[CPP - 1]], add=True)

        plsc.subcore_barrier()

        r0 = sid * rpt
        pltpu.sync_copy(acc.at[pl.ds(r0, rpt)], agg_hbm.at[cid, pl.ds(r0, rpt)])
        if with_deg:
            pltpu.sync_copy(dacc.at[pl.ds(r0, rpt)],
                            deg_hbm.at[cid, pl.ds(r0, rpt)])

    return sc_kernel


def _dot(a, b):
    return jnp.dot(a, b, preferred_element_type=f32)


def _tc_project_body(x_ref, wl_ref, wr_ref, b_ref, p_ref, r_ref):
    xb = x_ref[...]
    p_ref[...] = _dot(xb, wl_ref[...])
    r_ref[...] = _dot(xb, wr_ref[...]) + b_ref[...]


def _tc_combine_body(agg_ref, deg_ref, rp_ref, wl_ref, wr_ref, b_ref,
                     p_ref, r_ref):
    a = agg_ref[0] + agg_ref[1]
    dg = deg_ref[0, :, 0:1] + deg_ref[1, :, 0:1]
    h = jnp.maximum(a / jnp.maximum(dg, 1.0) + rp_ref[...], 0.0)
    p_ref[...] = _dot(h, wl_ref[...])
    r_ref[...] = _dot(h, wr_ref[...]) + b_ref[...]


def _tc_final_body(agg_ref, deg_ref, rp_ref, out_ref):
    a = agg_ref[0] + agg_ref[1]
    dg = deg_ref[0, :, 0:1] + deg_ref[1, :, 0:1]
    o = a / jnp.maximum(dg, 1.0) + rp_ref[...]
    m = jnp.max(o, axis=-1, keepdims=True)
    lse = jnp.log(jnp.sum(jnp.exp(o - m), axis=-1, keepdims=True)) + m
    out_ref[...] = o - lse


_row_spec = pl.BlockSpec((BR, D), lambda i: (i, 0))
_w_spec = pl.BlockSpec((D, D), lambda i: (0, 0))
_b_spec = pl.BlockSpec((1, D), lambda i: (0, 0))
_agg_spec = pl.BlockSpec((NC, BR, D), lambda i: (0, i, 0))
_deg_spec = pl.BlockSpec((NC, BR, 16), lambda i: (0, i, 0))
_pair_out = [jax.ShapeDtypeStruct((N_PAD, D), f32)] * 2

_tc_project = pl.pallas_call(
    _tc_project_body, grid=(N_PAD // BR,),
    in_specs=[_row_spec, _w_spec, _w_spec, _b_spec],
    out_specs=[_row_spec, _row_spec], out_shape=_pair_out)

_tc_combine = pl.pallas_call(
    _tc_combine_body, grid=(N_PAD // BR,),
    in_specs=[_agg_spec, _deg_spec, _row_spec, _w_spec, _w_spec, _b_spec],
    out_specs=[_row_spec, _row_spec], out_shape=_pair_out)

_tc_final = pl.pallas_call(
    _tc_final_body, grid=(N_PAD // BR,),
    in_specs=[_agg_spec, _deg_spec, _row_spec],
    out_specs=_row_spec, out_shape=jax.ShapeDtypeStruct((N_PAD, D), f32))


def kernel(x, adj_t, W1l, b1, W1r, W2l, b2, W2r, W3l, b3, W3r):
    adj3 = adj_t.reshape(2, E // K, K)
    b1r, b2r, b3r = (b.reshape(1, D) for b in (b1, b2, b3))

    p1, r1 = _tc_project(x, W1l, W1r, b1r)
    agg1, deg = _make_sc_segsum(True)(p1, adj3)
    p2, r2 = _tc_combine(agg1, deg, r1, W2l, W2r, b2r)
    agg2, = _make_sc_segsum(False)(p2, adj3)
    p3, r3 = _tc_combine(agg2, deg, r2, W3l, W3r, b3r)
    agg3, = _make_sc_segsum(False)(p3, adj3)
    return _tc_final(agg3, deg, r3)


# speedup vs baseline: 1.0163x; 1.0163x over previous
import functools

import jax
import jax.numpy as jnp
from jax import lax
from jax.experimental import pallas as pl
from jax.experimental.pallas import tpu as pltpu
from jax.experimental.pallas import tpu_sc as plsc

N = 10000
E = 320000
D = 128
N_PAD = N
BR = 1000
NC, NS = 2, 16
NW = NC * NS
EPT = E // NW
K = 80
NCH = EPT // K
PH = 5
CPP = NCH // PH
ZR = 25

f32 = jnp.float32


@functools.lru_cache(maxsize=None)
def _make_sc_segsum(with_deg: bool):
    out_type = [jax.ShapeDtypeStruct((NC, N_PAD, D), f32)]
    zr = ZR
    scratch = [
        pltpu.VMEM_SHARED((N_PAD, D), f32),
        pltpu.VMEM((zr, D), f32),
        pltpu.SemaphoreType.DMA,
        pltpu.VMEM((K, D), f32),
        pltpu.VMEM((K, D), f32),
        pltpu.VMEM((K, D), f32),
        pltpu.SemaphoreType.DMA,
        pltpu.SemaphoreType.DMA,
        pltpu.SemaphoreType.DMA,
        pltpu.SemaphoreType.DMA,
        pltpu.SemaphoreType.DMA,
        pltpu.SemaphoreType.DMA,
    ]
    if with_deg:
        out_type.append(jax.ShapeDtypeStruct((NC, N_PAD, 16), f32))
        scratch += [
            pltpu.VMEM((CPP, K), jnp.int32),
            pltpu.VMEM((CPP, K), jnp.int32),
            pltpu.SemaphoreType.DMA,
            pltpu.VMEM_SHARED((N_PAD, 16), f32),
            pltpu.VMEM((ZR, 16), f32),
            pltpu.VMEM((K, 16), f32),
            pltpu.SemaphoreType.DMA,
        ]
    else:
        scratch += [
            pltpu.VMEM((CPP, K), jnp.int32),
            pltpu.VMEM((CPP, K), jnp.int32),
            pltpu.VMEM((CPP, K), jnp.int32),
            pltpu.VMEM((CPP, K), jnp.int32),
            pltpu.SemaphoreType.DMA,
            pltpu.SemaphoreType.DMA,
        ]

    mesh = plsc.VectorSubcoreMesh(core_axis_name="c", subcore_axis_name="s",
                                  num_cores=NC, num_subcores=NS)

    @functools.partial(
        pl.kernel, out_type=out_type, mesh=mesh, scratch_types=scratch,
        compiler_params=pltpu.CompilerParams(use_tc_tiling_on_sc=False))
    def sc_kernel(p_hbm, adj_hbm, *refs):
        if with_deg:
            (agg_hbm, deg_hbm, acc, zbuf, zsem, rows0, rows1, rows2,
             sem0, sem1, sem2, ssem0, ssem1, ssem2,
             sidx0, didx0, isem0, dacc, dzbuf, ones, osem) = refs
            islot = [(sidx0, didx0, isem0)]
        else:
            (agg_hbm, acc, zbuf, zsem, rows0, rows1, rows2,
             sem0, sem1, sem2, ssem0, ssem1, ssem2,
             sidx0, didx0, sidx1, didx1, isem0, isem1) = refs
            islot = [(sidx0, didx0, isem0), (sidx1, didx1, isem1)]
        cid = lax.axis_index("c")
        sid = lax.axis_index("s")
        wid = sid * NC + cid
        rpt = N_PAD // NS
        nsl = len(islot)

        def issue_idx(ph):
            si, di, isem = islot[ph % nsl]
            r0 = wid * NCH + ph * CPP
            pltpu.async_copy(adj_hbm.at[0, pl.ds(r0, CPP)], si, isem)
            pltpu.async_copy(adj_hbm.at[1, pl.ds(r0, CPP)], di, isem)

        def wait_idx(ph):
            si, di, isem = islot[ph % nsl]
            pltpu.make_async_copy(adj_hbm.at[0, pl.ds(0, CPP)], si,
                                  isem).wait()
            pltpu.make_async_copy(adj_hbm.at[1, pl.ds(0, CPP)], di,
                                  isem).wait()

        issue_idx(0)

        z16 = jnp.zeros((16,), f32)

        def zrow(r, _):
            for j in range(D // 16):
                zbuf[r, pl.ds(j * 16, 16)] = z16
            return 0
        lax.fori_loop(0, zr, zrow, 0)

        def zacc(i, _):
            pltpu.async_copy(zbuf, acc.at[pl.ds(sid * rpt + i * zr, zr)],
                             zsem)
            return 0
        lax.fori_loop(0, rpt // zr, zacc, 0)

        if with_deg:
            def zdrow(r, _):
                dzbuf[r, :] = z16
                return 0
            lax.fori_loop(0, zr, zdrow, 0)

            def zdacc(i, _):
                pltpu.async_copy(dzbuf,
                                 dacc.at[pl.ds(sid * rpt + i * zr, zr)],
                                 zsem)
                return 0
            lax.fori_loop(0, rpt // zr, zdacc, 0)
            o16 = jnp.ones((16,), f32)

            def orow(r, _):
                ones[r, :] = o16
                return 0
            lax.fori_loop(0, K, orow, 0)

        wait_idx(0)
        sidx_p0 = islot[0][0]
        pltpu.async_copy(p_hbm.at[sidx_p0.at[0]], rows0, sem0)
        pltpu.async_copy(p_hbm.at[sidx_p0.at[1]], rows1, sem1)

        def zdrain(i, _):
            pltpu.make_async_copy(zbuf, acc.at[pl.ds(0, zr)], zsem).wait()
            return 0
        lax.fori_loop(0, rpt // zr, zdrain, 0)
        if with_deg:
            def zddrain(i, _):
                pltpu.make_async_copy(dzbuf, dacc.at[pl.ds(0, zr)],
                                      zsem).wait()
                return 0
            lax.fori_loop(0, rpt // zr, zddrain, 0)

        plsc.subcore_barrier()

        nb = (CPP - 1) // 3
        assert CPP == 3 * nb + 1
        for ph in range(PH):
            sidx, didx, _ = islot[ph % nsl]
            if nsl == 2 and ph + 1 < PH:
                issue_idx(ph + 1)
            if ph > 0:
                if nsl == 1:
                    issue_idx(ph)
                wait_idx(ph)
                pltpu.async_copy(p_hbm.at[sidx.at[0]], rows0, sem0)
                pltpu.async_copy(p_hbm.at[sidx.at[1]], rows1, sem1)

            def wait_g(rows, sem):
                pltpu.make_async_copy(p_hbm.at[sidx.at[0]], rows, sem).wait()

            def iscat(rows, c, ssem):
                pltpu.async_copy(rows, acc.at[didx.at[c]], ssem, add=True)
                if with_deg:
                    pltpu.async_copy(ones, dacc.at[didx.at[c]], osem,
                                     add=True)

            def wait_s(rows, ssem):
                pltpu.make_async_copy(rows, acc.at[didx.at[0]], ssem).wait()

            def ring(i, _):
                c = 3 * i
                wait_g(rows0, sem0)
                iscat(rows0, c, ssem0)
                pltpu.async_copy(p_hbm.at[sidx.at[c + 2]], rows2, sem2)
                wait_g(rows1, sem1)
                iscat(rows1, c + 1, ssem1)
                wait_s(rows0, ssem0)
                pltpu.async_copy(p_hbm.at[sidx.at[c + 3]], rows0, sem0)
                wait_g(rows2, sem2)
                iscat(rows2, c + 2, ssem2)
                wait_s(rows1, ssem1)

                @pl.when(c + 4 < CPP)
                def _():
                    pltpu.async_copy(p_hbm.at[sidx.at[c + 4]], rows1, sem1)
                wait_s(rows2, ssem2)
                return 0
            lax.fori_loop(0, nb, ring, 0)
            wait_g(rows0, sem0)
            pltpu.sync_copy(rows0, acc.at[didx.at[CPP - 1]], add=True)
            if with_deg:
                pltpu.sync_copy(ones, dacc.at[didx.at[CPP - 1]], add=True)

                def odrain(i, _):
                    pltpu.make_async_copy(ones, dacc.at[didx.at[0]],
                                          osem).wait()
                    return 0
                lax.fori_loop(0, 3 * nb, odrain, 0)

        plsc.subcore_barrier()

        r0 = sid * rpt
        pltpu.sync_copy(acc.at[pl.ds(r0, rpt)], agg_hbm.at[cid, pl.ds(r0, rpt)])
        if with_deg:
            pltpu.sync_copy(dacc.at[pl.ds(r0, rpt)],
                            deg_hbm.at[cid, pl.ds(r0, rpt)])

    return sc_kernel


def _dot(a, b):
    return jnp.dot(a, b, preferred_element_type=f32)


def _tc_project_body(x_ref, wl_ref, wr_ref, b_ref, p_ref, r_ref):
    xb = x_ref[...]
    p_ref[...] = _dot(xb, wl_ref[...])
    r_ref[...] = _dot(xb, wr_ref[...]) + b_ref[...]


def _tc_combine_body(agg_ref, deg_ref, rp_ref, wl_ref, wr_ref, b_ref,
                     p_ref, r_ref):
    a = agg_ref[0] + agg_ref[1]
    dg = deg_ref[0, :, 0:1] + deg_ref[1, :, 0:1]
    h = jnp.maximum(a / jnp.maximum(dg, 1.0) + rp_ref[...], 0.0)
    p_ref[...] = _dot(h, wl_ref[...])
    r_ref[...] = _dot(h, wr_ref[...]) + b_ref[...]


def _tc_final_body(agg_ref, deg_ref, rp_ref, out_ref):
    a = agg_ref[0] + agg_ref[1]
    dg = deg_ref[0, :, 0:1] + deg_ref[1, :, 0:1]
    o = a / jnp.maximum(dg, 1.0) + rp_ref[...]
    m = jnp.max(o, axis=-1, keepdims=True)
    lse = jnp.log(jnp.sum(jnp.exp(o - m), axis=-1, keepdims=True)) + m
    out_ref[...] = o - lse


_row_spec = pl.BlockSpec((BR, D), lambda i: (i, 0))
_w_spec = pl.BlockSpec((D, D), lambda i: (0, 0))
_b_spec = pl.BlockSpec((1, D), lambda i: (0, 0))
_agg_spec = pl.BlockSpec((NC, BR, D), lambda i: (0, i, 0))
_deg_spec = pl.BlockSpec((NC, BR, 16), lambda i: (0, i, 0))
_pair_out = [jax.ShapeDtypeStruct((N_PAD, D), f32)] * 2

_tc_project = pl.pallas_call(
    _tc_project_body, grid=(N_PAD // BR,),
    in_specs=[_row_spec, _w_spec, _w_spec, _b_spec],
    out_specs=[_row_spec, _row_spec], out_shape=_pair_out)

_tc_combine = pl.pallas_call(
    _tc_combine_body, grid=(N_PAD // BR,),
    in_specs=[_agg_spec, _deg_spec, _row_spec, _w_spec, _w_spec, _b_spec],
    out_specs=[_row_spec, _row_spec], out_shape=_pair_out)

_tc_final = pl.pallas_call(
    _tc_final_body, grid=(N_PAD // BR,),
    in_specs=[_agg_spec, _deg_spec, _row_spec],
    out_specs=_row_spec, out_shape=jax.ShapeDtypeStruct((N_PAD, D), f32))


def kernel(x, adj_t, W1l, b1, W1r, W2l, b2, W2r, W3l, b3, W3r):
    adj3 = adj_t.reshape(2, E // K, K)
    b1r, b2r, b3r = (b.reshape(1, D) for b in (b1, b2, b3))

    p1, r1 = _tc_project(x, W1l, W1r, b1r)
    agg1, deg = _make_sc_segsum(True)(p1, adj3)
    p2, r2 = _tc_combine(agg1, deg, r1, W2l, W2r, b2r)
    agg2, = _make_sc_segsum(False)(p2, adj3)
    p3, r3 = _tc_combine(agg2, deg, r2, W3l, W3r, b3r)
    agg3, = _make_sc_segsum(False)(p3, adj3)
    return _tc_final(agg3, deg, r3)


# speedup vs baseline: 1.0205x; 1.0042x over previous
import functools

import jax
import jax.numpy as jnp
from jax import lax
from jax.experimental import pallas as pl
from jax.experimental.pallas import tpu as pltpu
from jax.experimental.pallas import tpu_sc as plsc

N = 10000
E = 320000
D = 128
N_PAD = N
BR = 1000
NC, NS = 2, 16
NW = NC * NS
EPT = E // NW
K = 80
NCH = EPT // K
PH = 5
CPP = NCH // PH
ZR = 25

f32 = jnp.float32


@functools.lru_cache(maxsize=None)
def _make_sc_segsum(with_deg: bool):
    out_type = [jax.ShapeDtypeStruct((NC, N_PAD, D), f32)]
    zr = ZR
    scratch = [
        pltpu.VMEM_SHARED((N_PAD, D), f32),
        pltpu.VMEM((zr, D), f32),
        pltpu.SemaphoreType.DMA,
        pltpu.VMEM((K, D), f32),
        pltpu.VMEM((K, D), f32),
        pltpu.VMEM((K, D), f32),
        pltpu.SemaphoreType.DMA,
        pltpu.SemaphoreType.DMA,
        pltpu.SemaphoreType.DMA,
        pltpu.SemaphoreType.DMA,
        pltpu.SemaphoreType.DMA,
        pltpu.SemaphoreType.DMA,
    ]
    if with_deg:
        out_type.append(jax.ShapeDtypeStruct((NC, N_PAD, 16), f32))
        scratch += [
            pltpu.VMEM((CPP, K), jnp.int32),
            pltpu.VMEM((CPP, K), jnp.int32),
            pltpu.SemaphoreType.DMA,
            pltpu.VMEM_SHARED((N_PAD, 16), f32),
            pltpu.VMEM((ZR, 16), f32),
            pltpu.VMEM((K, 16), f32),
            pltpu.SemaphoreType.DMA,
        ]
    else:
        scratch += [
            pltpu.VMEM((CPP, K), jnp.int32),
            pltpu.VMEM((CPP, K), jnp.int32),
            pltpu.VMEM((CPP, K), jnp.int32),
            pltpu.VMEM((CPP, K), jnp.int32),
            pltpu.SemaphoreType.DMA,
            pltpu.SemaphoreType.DMA,
        ]

    mesh = plsc.VectorSubcoreMesh(core_axis_name="c", subcore_axis_name="s",
                                  num_cores=NC, num_subcores=NS)

    @functools.partial(
        pl.kernel, out_type=out_type, mesh=mesh, scratch_types=scratch,
        compiler_params=pltpu.CompilerParams(use_tc_tiling_on_sc=False))
    def sc_kernel(p_hbm, adj_hbm, *refs):
        if with_deg:
            (agg_hbm, deg_hbm, acc, zbuf, zsem, rows0, rows1, rows2,
             sem0, sem1, sem2, ssem0, ssem1, ssem2,
             sidx0, didx0, isem0, dacc, dzbuf, ones, osem) = refs
            islot = [(sidx0, didx0, isem0)]
        else:
            (agg_hbm, acc, zbuf, zsem, rows0, rows1, rows2,
             sem0, sem1, sem2, ssem0, ssem1, ssem2,
             sidx0, didx0, sidx1, didx1, isem0, isem1) = refs
            islot = [(sidx0, didx0, isem0), (sidx1, didx1, isem1)]
        cid = lax.axis_index("c")
        sid = lax.axis_index("s")
        wid = sid * NC + cid
        rpt = N_PAD // NS
        nsl = len(islot)

        def issue_idx(ph):
            si, di, isem = islot[ph % nsl]
            r0 = wid * NCH + ph * CPP
            pltpu.async_copy(adj_hbm.at[0, pl.ds(r0, CPP)], si, isem)
            pltpu.async_copy(adj_hbm.at[1, pl.ds(r0, CPP)], di, isem)

        def wait_idx(ph):
            si, di, isem = islot[ph % nsl]
            pltpu.make_async_copy(adj_hbm.at[0, pl.ds(0, CPP)], si,
                                  isem).wait()
            pltpu.make_async_copy(adj_hbm.at[1, pl.ds(0, CPP)], di,
                                  isem).wait()

        issue_idx(0)

        z16 = jnp.zeros((16,), f32)

        def zrow(r, _):
            for j in range(D // 16):
                zbuf[r, pl.ds(j * 16, 16)] = z16
            return 0
        lax.fori_loop(0, zr, zrow, 0)

        def zacc(i, _):
            pltpu.async_copy(zbuf, acc.at[pl.ds(sid * rpt + i * zr, zr)],
                             zsem)
            return 0
        lax.fori_loop(0, rpt // zr, zacc, 0)

        if with_deg:
            def zdrow(r, _):
                dzbuf[r, :] = z16
                return 0
            lax.fori_loop(0, zr, zdrow, 0)

            def zdacc(i, _):
                pltpu.async_copy(dzbuf,
                                 dacc.at[pl.ds(sid * rpt + i * zr, zr)],
                                 zsem)
                return 0
            lax.fori_loop(0, rpt // zr, zdacc, 0)
            o16 = jnp.ones((16,), f32)

            def orow(r, _):
                ones[r, :] = o16
                return 0
            lax.fori_loop(0, K, orow, 0)

        wait_idx(0)
        sidx_p0 = islot[0][0]
        pltpu.async_copy(p_hbm.at[sidx_p0.at[0]], rows0, sem0)
        pltpu.async_copy(p_hbm.at[sidx_p0.at[1]], rows1, sem1)

        def zdrain(i, _):
            pltpu.make_async_copy(zbuf, acc.at[pl.ds(0, zr)], zsem).wait()
            return 0
        lax.fori_loop(0, rpt // zr, zdrain, 0)
        if with_deg:
            def zddrain(i, _):
                pltpu.make_async_copy(dzbuf, dacc.at[pl.ds(0, zr)],
                                      zsem).wait()
                return 0
            lax.fori_loop(0, rpt // zr, zddrain, 0)

        plsc.subcore_barrier()

        nb = (CPP - 1) // 3
        assert CPP == 3 * nb + 1
        for ph in range(PH):
            sidx, didx, _ = islot[ph % nsl]
            if nsl == 2 and ph + 1 < PH:
                issue_idx(ph + 1)
            if ph > 0:
                if nsl == 1:
                    issue_idx(ph)
                wait_idx(ph)
                pltpu.async_copy(p_hbm.at[sidx.at[0]], rows0, sem0)
                pltpu.async_copy(p_hbm.at[sidx.at[1]], rows1, sem1)

            def wait_g(rows, sem):
                pltpu.make_async_copy(p_hbm.at[sidx.at[0]], rows, sem).wait()

            def iscat(rows, c, ssem):
                pltpu.async_copy(rows, acc.at[didx.at[c]], ssem, add=True)
                if with_deg:
                    pltpu.async_copy(ones, dacc.at[didx.at[c]], osem,
                                     add=True)

            def wait_s(rows, ssem):
                pltpu.make_async_copy(rows, acc.at[didx.at[0]], ssem).wait()

            def ring(i, _):
                c = 3 * i
                wait_g(rows0, sem0)
                iscat(rows0, c, ssem0)
                pltpu.async_copy(p_hbm.at[sidx.at[c + 2]], rows2, sem2)
                wait_g(rows1, sem1)
                iscat(rows1, c + 1, ssem1)
                wait_s(rows0, ssem0)
                pltpu.async_copy(p_hbm.at[sidx.at[c + 3]], rows0, sem0)
                wait_g(rows2, sem2)
                iscat(rows2, c + 2, ssem2)
                wait_s(rows1, ssem1)

                @pl.when(c + 4 < CPP)
                def _():
                    pltpu.async_copy(p_hbm.at[sidx.at[c + 4]], rows1, sem1)
                wait_s(rows2, ssem2)
                return 0
            lax.fori_loop(0, nb, ring, 0)
            wait_g(rows0, sem0)
            pltpu.sync_copy(rows0, acc.at[didx.at[CPP - 1]], add=True)
            if with_deg:
                pltpu.sync_copy(ones, dacc.at[didx.at[CPP - 1]], add=True)

                def odrain(i, _):
                    pltpu.make_async_copy(ones, dacc.at[didx.at[0]],
                                          osem).wait()
                    return 0
                lax.fori_loop(0, 3 * nb, odrain, 0)

        plsc.subcore_barrier()

        r0 = sid * rpt
        pltpu.sync_copy(acc.at[pl.ds(r0, rpt)], agg_hbm.at[cid, pl.ds(r0, rpt)])
        if with_deg:
            pltpu.sync_copy(dacc.at[pl.ds(r0, rpt)],
                            deg_hbm.at[cid, pl.ds(r0, rpt)])

    return sc_kernel


def _dot(a, b):
    return jnp.dot(a, b, preferred_element_type=f32)


def _tc_pproj_body(x_ref, wl_ref, p_ref):
    p_ref[...] = _dot(x_ref[...], wl_ref[...])


def _tc_rproj_body(x_ref, wr_ref, b_ref, r_ref):
    r_ref[...] = _dot(x_ref[...], wr_ref[...]) + b_ref[...]


def _tc_combine_body(agg_ref, deg_ref, rp_ref, wl_ref, wr_ref, b_ref,
                     p_ref, r_ref):
    a = agg_ref[0] + agg_ref[1]
    dg = deg_ref[0, :, 0:1] + deg_ref[1, :, 0:1]
    h = jnp.maximum(a / jnp.maximum(dg, 1.0) + rp_ref[...], 0.0)
    p_ref[...] = _dot(h, wl_ref[...])
    r_ref[...] = _dot(h, wr_ref[...]) + b_ref[...]


def _tc_final_body(agg_ref, deg_ref, rp_ref, out_ref):
    a = agg_ref[0] + agg_ref[1]
    dg = deg_ref[0, :, 0:1] + deg_ref[1, :, 0:1]
    o = a / jnp.maximum(dg, 1.0) + rp_ref[...]
    m = jnp.max(o, axis=-1, keepdims=True)
    lse = jnp.log(jnp.sum(jnp.exp(o - m), axis=-1, keepdims=True)) + m
    out_ref[...] = o - lse


_row_spec = pl.BlockSpec((BR, D), lambda i: (i, 0))
_w_spec = pl.BlockSpec((D, D), lambda i: (0, 0))
_b_spec = pl.BlockSpec((1, D), lambda i: (0, 0))
_agg_spec = pl.BlockSpec((NC, BR, D), lambda i: (0, i, 0))
_deg_spec = pl.BlockSpec((NC, BR, 16), lambda i: (0, i, 0))
_pair_out = [jax.ShapeDtypeStruct((N_PAD, D), f32)] * 2

_tc_pproj = pl.pallas_call(
    _tc_pproj_body, grid=(N_PAD // BR,),
    in_specs=[_row_spec, _w_spec],
    out_specs=_row_spec, out_shape=jax.ShapeDtypeStruct((N_PAD, D), f32))

_tc_rproj = pl.pallas_call(
    _tc_rproj_body, grid=(N_PAD // BR,),
    in_specs=[_row_spec, _w_spec, _b_spec],
    out_specs=_row_spec, out_shape=jax.ShapeDtypeStruct((N_PAD, D), f32))

_tc_combine = pl.pallas_call(
    _tc_combine_body, grid=(N_PAD // BR,),
    in_specs=[_agg_spec, _deg_spec, _row_spec, _w_spec, _w_spec, _b_spec],
    out_specs=[_row_spec, _row_spec], out_shape=_pair_out)

_tc_final = pl.pallas_call(
    _tc_final_body, grid=(N_PAD // BR,),
    in_specs=[_agg_spec, _deg_spec, _row_spec],
    out_specs=_row_spec, out_shape=jax.ShapeDtypeStruct((N_PAD, D), f32))


def kernel(x, adj_t, W1l, b1, W1r, W2l, b2, W2r, W3l, b3, W3r):
    adj3 = adj_t.reshape(2, E // K, K)
    b1r, b2r, b3r = (b.reshape(1, D) for b in (b1, b2, b3))

    p1 = _tc_pproj(x, W1l)
    agg1, deg = _make_sc_segsum(True)(p1, adj3)
    r1 = _tc_rproj(x, W1r, b1r)
    p2, r2 = _tc_combine(agg1, deg, r1, W2l, W2r, b2r)
    agg2, = _make_sc_segsum(False)(p2, adj3)
    p3, r3 = _tc_combine(agg2, deg, r2, W3l, W3r, b3r)
    agg3, = _make_sc_segsum(False)(p3, adj3)
    return _tc_final(agg3, deg, r3)
